# baseline (speedup 1.0000x reference)
import functools

import jax
import jax.numpy as jnp
import numpy as np
from jax import lax
from jax.experimental import pallas as pl
from jax.experimental.pallas import tpu as pltpu
from jax.experimental.pallas import tpu_sc as plsc

_NUM_LEVELS = 8
_VOCAB = 1024
_EMBED_DIM = 64

_C = 128
_G = 2
_M = _C * _G
_NBUF = 4


@functools.lru_cache(maxsize=None)
def _build(num_rows):
    info = plsc.get_sparse_core_info()
    nc, ns = info.num_cores, info.num_subcores
    nw = nc * ns
    rows_per_w = num_rows // nw
    idx_rows_per_w = rows_per_w // _C
    nm = rows_per_w // _M
    assert nm >= 4 and (nm - 4) % _NBUF == 0

    mesh = plsc.VectorSubcoreMesh(core_axis_name="c", subcore_axis_name="s")

    @functools.partial(
        pl.kernel,
        mesh=mesh,
        out_type=jax.ShapeDtypeStruct((num_rows, _EMBED_DIM), jnp.float32),
        compiler_params=pltpu.CompilerParams(use_tc_tiling_on_sc=False),
        scratch_types=[
            pltpu.VMEM((idx_rows_per_w, _C), jnp.int32),
            pltpu.VMEM((_NBUF, _M, _EMBED_DIM), jnp.float32),
        ]
        + [pltpu.SemaphoreType.DMA] * (2 * _NBUF),
    )
    def k(idx_hbm, table_hbm, out_hbm, idx_v, rows_v, *sems):
        gsem = sems[:_NBUF]
        wsem = sems[_NBUF:]
        wid = lax.axis_index("s") * nc + lax.axis_index("c")
        row_base = wid * rows_per_w

        pltpu.sync_copy(
            idx_hbm.at[pl.ds(wid * idx_rows_per_w, idx_rows_per_w)], idx_v
        )

        def g_descs(mc, buf):
            return [
                pltpu.make_async_copy(
                    table_hbm.at[idx_v.at[mc * _G + g]],
                    rows_v.at[buf, pl.ds(g * _C, _C)],
                    gsem[buf],
                )
                for g in range(_G)
            ]

        def w_desc(mc, buf):
            return pltpu.make_async_copy(
                rows_v.at[buf],
                out_hbm.at[pl.ds(row_base + mc * _M, _M)],
                wsem[buf],
            )

        def start_g(mc, buf):
            for d in g_descs(mc, buf):
                d.start()

        def wait_g(mc, buf):
            for d in g_descs(mc, buf):
                d.wait()

        for mc in range(_NBUF):
            start_g(mc, mc)
        wait_g(0, 0)
        w_desc(0, 0).start()
        wait_g(1, 1)
        w_desc(1, 1).start()

        def body(j, carry):
            for b4 in range(_NBUF):
                mc = 2 + j * _NBUF + b4
                buf = (2 + b4) % _NBUF
                nbuf = b4 % _NBUF
                wait_g(mc, buf)
                w_desc(mc, buf).start()
                w_desc(mc - 2, nbuf).wait()
                start_g(mc + 2, nbuf)
            return carry

        lax.fori_loop(0, (nm - 4) // _NBUF, body, 0)

        for mc in (nm - 2, nm - 1):
            buf = mc % _NBUF
            wait_g(mc, buf)
            w_desc(mc, buf).start()
            w_desc(mc - 2, (mc - 2) % _NBUF).wait()
        w_desc(nm - 2, (nm - 2) % _NBUF).wait()
        w_desc(nm - 1, (nm - 1) % _NBUF).wait()

    return k


def _perm_matrix():
    m = np.zeros((64, 64), np.float32)
    for p in range(64):
        ct, r = divmod(p, 16)
        tok, par = divmod(r, 2)
        m[tok * 8 + 2 * ct + par, p] = 1.0
    return jnp.asarray(m)


def _level_offsets():
    return jnp.asarray(
        [(2 * (p // 16) + p % 2) * _VOCAB for p in range(64)], jnp.float32
    )


_TBLK = 64


def _untile(x4):
    ntr = x4.shape[0]
    grid = (ntr // _TBLK, 4)

    def body(x_ref, o_ref):
        o_ref[...] = x_ref[...].reshape(_TBLK * 8, 128)

    return pl.pallas_call(
        body,
        grid=grid,
        in_specs=[pl.BlockSpec((_TBLK, 1, 8, 128), lambda i, ct: (i, ct, 0, 0))],
        out_specs=pl.BlockSpec((_TBLK * 8, 128), lambda i, ct: (i, ct)),
        out_shape=jax.ShapeDtypeStruct((ntr * 8, 512), jnp.float32),
    )(x4)


def kernel(codes, tables):
    b, l, q = codes.shape
    _, v, d = tables.shape
    n = b * l * q
    blocks = codes.reshape(n // 64, 64).astype(jnp.float32)
    permuted = (
        jnp.dot(blocks, _perm_matrix(), precision=lax.Precision.HIGHEST)
        + _level_offsets()
    )
    idx = permuted.astype(jnp.int32).reshape(n // _C, _C)
    out = _build(n)(idx, tables.reshape(q * v, d))
    y = _untile(out.reshape(n // 64, 4, 8, 2 * d))
    return y.reshape(b, l, q * d)


# speedup vs baseline: 1.5689x; 1.5689x over previous
import functools

import jax
import jax.numpy as jnp
from jax import lax
from jax.experimental import pallas as pl
from jax.experimental.pallas import tpu as pltpu
from jax.experimental.pallas import tpu_sc as plsc

_NUM_LEVELS = 8
_VOCAB = 1024
_EMBED_DIM = 64

_C = 128
_G = 2
_M = _C * _G
_NBUF = 6
_GLA = 3
_WLA = _NBUF - _GLA


@functools.lru_cache(maxsize=None)
def _build(num_rows):
    info = plsc.get_sparse_core_info()
    nc, ns = info.num_cores, info.num_subcores
    nw = nc * ns
    rows_per_w = num_rows // nw
    idx_rows_per_w = rows_per_w // _C
    nm = rows_per_w // _M

    mesh = plsc.VectorSubcoreMesh(core_axis_name="c", subcore_axis_name="s")

    @functools.partial(
        pl.kernel,
        mesh=mesh,
        out_type=jax.ShapeDtypeStruct((num_rows, _EMBED_DIM), jnp.float32),
        compiler_params=pltpu.CompilerParams(use_tc_tiling_on_sc=False),
        scratch_types=[
            pltpu.VMEM((idx_rows_per_w, _C), jnp.int32),
            pltpu.VMEM((_NBUF, _M, _EMBED_DIM), jnp.float32),
        ]
        + [pltpu.SemaphoreType.DMA] * (2 * _NBUF),
    )
    def k(idx_hbm, table_hbm, out_hbm, idx_v, rows_v, *sems):
        gsem = sems[:_NBUF]
        wsem = sems[_NBUF:]
        wid = lax.axis_index("s") * nc + lax.axis_index("c")
        row_base = wid * rows_per_w

        pltpu.sync_copy(
            idx_hbm.at[pl.ds(wid * idx_rows_per_w, idx_rows_per_w)], idx_v
        )

        def g_descs(mc, buf):
            return [
                pltpu.make_async_copy(
                    table_hbm.at[idx_v.at[mc * _G + g]],
                    rows_v.at[buf, pl.ds(g * _C, _C)],
                    gsem[buf],
                )
                for g in range(_G)
            ]

        def w_desc(mc, buf):
            return pltpu.make_async_copy(
                rows_v.at[buf],
                out_hbm.at[pl.ds(row_base + mc * _M, _M)],
                wsem[buf],
            )

        for mc in range(_GLA):
            for d in g_descs(mc, mc % _NBUF):
                d.start()
        for mc in range(nm):
            buf = mc % _NBUF
            for d in g_descs(mc, buf):
                d.wait()
            w_desc(mc, buf).start()
            if mc >= _WLA:
                w_desc(mc - _WLA, (mc - _WLA) % _NBUF).wait()
            nxt = mc + _GLA
            if nxt < nm:
                for d in g_descs(nxt, nxt % _NBUF):
                    d.start()
        for mc in range(nm - _WLA, nm):
            w_desc(mc, mc % _NBUF).wait()

    return k


def kernel(codes, tables):
    b, l, q = codes.shape
    _, v, d = tables.shape
    n = b * l * q
    adj = codes + jnp.arange(q, dtype=codes.dtype) * v
    idx = adj.reshape(n // _C, _C)
    out = _build(n)(idx, tables.reshape(q * v, d))
    return out.reshape(b, l, q * d)


# speedup vs baseline: 1.5844x; 1.0099x over previous
import functools

import jax
import jax.numpy as jnp
from jax import lax
from jax.experimental import pallas as pl
from jax.experimental.pallas import tpu as pltpu
from jax.experimental.pallas import tpu_sc as plsc

_NUM_LEVELS = 8
_VOCAB = 1024
_EMBED_DIM = 64

_C = 128
_G = 2
_M = _C * _G
_NBUF = 4


@functools.lru_cache(maxsize=None)
def _build(num_rows):
    info = plsc.get_sparse_core_info()
    nc, ns = info.num_cores, info.num_subcores
    nw = nc * ns
    rows_per_w = num_rows // nw
    idx_rows_per_w = rows_per_w // _C
    nm = rows_per_w // _M
    assert nm >= 4 and (nm - 4) % _NBUF == 0

    mesh = plsc.VectorSubcoreMesh(core_axis_name="c", subcore_axis_name="s")

    @functools.partial(
        pl.kernel,
        mesh=mesh,
        out_type=jax.ShapeDtypeStruct((num_rows, _EMBED_DIM), jnp.float32),
        compiler_params=pltpu.CompilerParams(use_tc_tiling_on_sc=False),
        scratch_types=[
            pltpu.VMEM((idx_rows_per_w, _C), jnp.int32),
            pltpu.VMEM((_NBUF, _M, _EMBED_DIM), jnp.float32),
        ]
        + [pltpu.SemaphoreType.DMA] * (2 * _NBUF),
    )
    def k(idx_hbm, table_hbm, out_hbm, idx_v, rows_v, *sems):
        gsem = sems[:_NBUF]
        wsem = sems[_NBUF:]
        wid = lax.axis_index("s") * nc + lax.axis_index("c")
        row_base = wid * rows_per_w

        pltpu.sync_copy(
            idx_hbm.at[pl.ds(wid * idx_rows_per_w, idx_rows_per_w)], idx_v
        )

        def g_descs(mc, buf):
            return [
                pltpu.make_async_copy(
                    table_hbm.at[idx_v.at[mc * _G + g]],
                    rows_v.at[buf, pl.ds(g * _C, _C)],
                    gsem[buf],
                )
                for g in range(_G)
            ]

        def w_desc(mc, buf):
            return pltpu.make_async_copy(
                rows_v.at[buf],
                out_hbm.at[pl.ds(row_base + mc * _M, _M)],
                wsem[buf],
            )

        def start_g(mc, buf):
            for d in g_descs(mc, buf):
                d.start()

        def wait_g(mc, buf):
            for d in g_descs(mc, buf):
                d.wait()

        for mc in range(_NBUF):
            start_g(mc, mc)
        wait_g(0, 0)
        w_desc(0, 0).start()
        wait_g(1, 1)
        w_desc(1, 1).start()

        def body(j, carry):
            for b4 in range(_NBUF):
                mc = 2 + j * _NBUF + b4
                buf = (2 + b4) % _NBUF
                nbuf = b4 % _NBUF
                wait_g(mc, buf)
                w_desc(mc, buf).start()
                w_desc(mc - 2, nbuf).wait()
                start_g(mc + 2, nbuf)
            return carry

        lax.fori_loop(0, (nm - 4) // _NBUF, body, 0)

        for mc in (nm - 2, nm - 1):
            buf = mc % _NBUF
            wait_g(mc, buf)
            w_desc(mc, buf).start()
            w_desc(mc - 2, (mc - 2) % _NBUF).wait()
        w_desc(nm - 2, (nm - 2) % _NBUF).wait()
        w_desc(nm - 1, (nm - 1) % _NBUF).wait()

    return k


def kernel(codes, tables):
    b, l, q = codes.shape
    _, v, d = tables.shape
    n = b * l * q
    adj = codes + jnp.arange(q, dtype=codes.dtype) * v
    idx = adj.reshape(n // _C, _C)
    out = _build(n)(idx, tables.reshape(q * v, d))
    return out.reshape(b, l, q * d)
